# baseline (speedup 1.0000x reference)
import jax
import jax.numpy as jnp
from jax import lax
from jax.experimental import pallas as pl
from jax.experimental.pallas import tpu as pltpu
from jax.experimental.pallas import tpu_sc as plsc

_D = 64
_BATCH, _HIST = 4096, 200
_NC, _NS = 2, 16
_NW = _NC * _NS
_BPW = _BATCH // _NW
_NBUF = 4
_NSTEPS = _HIST // _NBUF


def _gather_body(table_hbm, idx_hbm, out_hbm, idx_v, rows_v, *sems):
    g_sems, o_sems = sems[:_NBUF], sems[_NBUF:]
    wid = lax.axis_index("s") * _NC + lax.axis_index("c")
    base = wid * _BPW
    pltpu.sync_copy(idx_hbm.at[:, pl.ds(base, _BPW)], idx_v)

    def gather(h, b):
        return pltpu.make_async_copy(
            table_hbm.at[idx_v.at[h]], rows_v.at[b], g_sems[b]
        )

    def writeback(h, b):
        return pltpu.make_async_copy(
            rows_v.at[b], out_hbm.at[pl.ds(base, _BPW), h], o_sems[b]
        )

    for b in range(_NBUF):
        gather(b, b).start()

    def step(s, carry):
        h0 = s * _NBUF
        for b in range(_NBUF):
            gather(h0 + b, b).wait()
            writeback(h0 + b, b).start()
        for b in range(_NBUF):
            writeback(h0 + b, b).wait()
            gather(h0 + _NBUF + b, b).start()
        return carry

    lax.fori_loop(0, _NSTEPS - 1, step, 0)

    h0 = (_NSTEPS - 1) * _NBUF
    for b in range(_NBUF):
        gather(h0 + b, b).wait()
        writeback(h0 + b, b).start()
    for b in range(_NBUF):
        writeback(h0 + b, b).wait()


@jax.jit
def kernel(x, weight):
    idx_t = jnp.transpose(x.astype(jnp.int32))
    mesh = plsc.VectorSubcoreMesh(core_axis_name="c", subcore_axis_name="s")
    return pl.kernel(
        _gather_body,
        out_type=jax.ShapeDtypeStruct((_BATCH, _HIST, _D), jnp.float32),
        mesh=mesh,
        scratch_types=[
            pltpu.VMEM((_HIST, _BPW), jnp.int32),
            pltpu.VMEM((_NBUF, _BPW, _D), jnp.float32),
        ] + [pltpu.SemaphoreType.DMA] * (2 * _NBUF),
        compiler_params=pltpu.CompilerParams(use_tc_tiling_on_sc=False),
    )(weight, idx_t)


# speedup vs baseline: 1.3295x; 1.3295x over previous
import jax
import jax.numpy as jnp
from jax import lax
from jax.experimental import pallas as pl
from jax.experimental.pallas import tpu as pltpu
from jax.experimental.pallas import tpu_sc as plsc

_D = 64
_BATCH, _HIST = 4096, 200
_NC, _NS = 2, 16
_NW = _NC * _NS
_BPW = _BATCH // _NW
_NBUF = 4
_NSTEPS = _HIST // _NBUF


def _gather_body(table_hbm, idx_hbm, out_hbm, idx_v, rows_v, *sems):
    g_sems, o_sems = sems[:_NBUF], sems[_NBUF:]
    wid = lax.axis_index("s") * _NC + lax.axis_index("c")
    base = wid * _BPW
    pltpu.sync_copy(idx_hbm.at[:, pl.ds(base, _BPW)], idx_v)

    def gather(h, b):
        return pltpu.make_async_copy(
            table_hbm.at[idx_v.at[h]], rows_v.at[b], g_sems[b]
        )

    def writeback(h, b):
        return pltpu.make_async_copy(
            rows_v.at[b],
            out_hbm.at[pl.ds(base, _BPW), h // 8, h % 8, pl.ds(0, _D)],
            o_sems[b],
        )

    for b in range(_NBUF):
        gather(b, b).start()

    def step(s, carry):
        h0 = s * _NBUF
        for b in range(_NBUF):
            gather(h0 + b, b).wait()
            writeback(h0 + b, b).start()
        for b in range(_NBUF):
            writeback(h0 + b, b).wait()
            gather(h0 + _NBUF + b, b).start()
        return carry

    lax.fori_loop(0, _NSTEPS - 1, step, 0)

    h0 = (_NSTEPS - 1) * _NBUF
    for b in range(_NBUF):
        gather(h0 + b, b).wait()
        writeback(h0 + b, b).start()
    for b in range(_NBUF):
        writeback(h0 + b, b).wait()


@jax.jit
def kernel(x, weight):
    idx_t = jnp.transpose(x.astype(jnp.int32))
    mesh = plsc.VectorSubcoreMesh(core_axis_name="c", subcore_axis_name="s")
    out4 = pl.kernel(
        _gather_body,
        out_type=jax.ShapeDtypeStruct((_BATCH, _HIST // 8, 8, 2 * _D), jnp.float32),
        mesh=mesh,
        scratch_types=[
            pltpu.VMEM((_HIST, _BPW), jnp.int32),
            pltpu.VMEM((_NBUF, _BPW, _D), jnp.float32),
        ] + [pltpu.SemaphoreType.DMA] * (2 * _NBUF),
        compiler_params=pltpu.CompilerParams(use_tc_tiling_on_sc=False),
    )(weight, idx_t)
    return out4.reshape(_BATCH, _HIST, 2 * _D)[:, :, :_D]
